# baseline (speedup 1.0000x reference)
import functools

import jax
import jax.numpy as jnp
from jax import lax
from jax.experimental import pallas as pl
from jax.experimental.pallas import tpu as pltpu
from jax.experimental.pallas import tpu_sc as plsc

_B = 16384
_NEG = 100
_LANES = 16
_PITCH = 128
_SENT = 112
_NC = 2
_NS = 16
_NW = _NC * _NS
_ROWS_W = _B // _NW
_GROUPS = _ROWS_W // _LANES
_BIG = 3e38


def _sc_body(sim_hbm, len_hbm, lab_hbm, out_hbm, sim_v, len_v, lab_v, res_v):
    wid = lax.axis_index("s") * _NC + lax.axis_index("c")
    base_row = wid * _ROWS_W
    pltpu.sync_copy(sim_hbm.at[pl.ds(base_row, _ROWS_W), :], sim_v)
    pltpu.sync_copy(len_hbm.at[pl.ds(base_row, _ROWS_W)], len_v)
    pltpu.sync_copy(lab_hbm.at[pl.ds(base_row, _ROWS_W)], lab_v)

    lane = lax.iota(jnp.int32, _LANES)
    zero = jnp.zeros((_LANES,), jnp.float32)
    bigv = jnp.full((_LANES,), jnp.float32(_BIG))
    c001 = jnp.full((_LANES,), jnp.float32(0.001))

    def fill(g, carry):
        rows = g * _LANES + lane
        for k in range(_PITCH - _SENT):
            cols = jnp.full((_LANES,), _SENT + k, jnp.int32)
            plsc.store_scatter(sim_v, [rows, cols], bigv)
        return carry

    lax.fori_loop(0, _GROUPS, fill, 0)

    def group_body(g, grand):
        rows = g * _LANES + lane
        l_u = len_v[pl.ds(g * _LANES, _LANES)].astype(jnp.uint32)

        def tbody(t, c):
            s0, s1, m0, m1 = c
            xs = []
            for k in range(4):
                ct = lane + (t * 4 + k)
                c_raw = (ct & (_PITCH - 1)).astype(jnp.uint32)
                valid = c_raw < l_u
                c_idx = jnp.where(valid, c_raw,
                                  c_raw | jnp.uint32(_SENT)).astype(jnp.int32)
                xs.append(plsc.load_gather(sim_v, [rows, c_idx]))
            s0 = (s0 + jnp.minimum(xs[0], c001)) + jnp.minimum(xs[2], c001)
            s1 = (s1 + jnp.minimum(xs[1], c001)) + jnp.minimum(xs[3], c001)
            m0 = jnp.minimum(jnp.minimum(m0, xs[0]), xs[2])
            m1 = jnp.minimum(jnp.minimum(m1, xs[1]), xs[3])
            return (s0, s1, m0, m1)

        s0, s1, m0, m1 = lax.fori_loop(
            0, _PITCH // 4, tbody, (zero, zero, bigv, bigv))
        s_vec = s0 + s1
        min_vec = jnp.minimum(m0, m1)
        relu_sum = jnp.float32(0.001 * _PITCH) - s_vec
        l_true = len_v[pl.ds(g * _LANES, _LANES)]
        lab = lab_v[pl.ds(g * _LANES, _LANES)]
        mean = relu_sum / l_true.astype(jnp.float32)
        u = min_vec + jnp.float32(0.15)
        mis = jnp.where(u >= 0, u, u * jnp.float32(0.01))
        return grand + jnp.where(lab == 1, mean, mis)

    grand = lax.fori_loop(0, _GROUPS, group_body, zero)
    res_v[...] = jnp.where(lane == 0, jnp.sum(grand), jnp.float32(0.0))
    pltpu.sync_copy(res_v, out_hbm.at[wid])


@jax.jit
def _mnloss_sc(sim_neg, lengths, labels):
    mesh = plsc.VectorSubcoreMesh(core_axis_name="c", subcore_axis_name="s")
    run = functools.partial(
        pl.kernel,
        mesh=mesh,
        compiler_params=pltpu.CompilerParams(needs_layout_passes=False),
        out_type=jax.ShapeDtypeStruct((_NW, _LANES), jnp.float32),
        scratch_types=[
            pltpu.VMEM((_ROWS_W, _NEG), jnp.float32),
            pltpu.VMEM((_ROWS_W,), jnp.int32),
            pltpu.VMEM((_ROWS_W,), jnp.int32),
            pltpu.VMEM((_LANES,), jnp.float32),
        ],
    )(_sc_body)
    return run(sim_neg, lengths, labels)


def kernel(sim_neg, train_mn_label, mn_length):
    partials = _mnloss_sc(sim_neg, mn_length, train_mn_label)
    return jnp.sum(partials).reshape(1)
